# baseline (speedup 1.0000x reference)
import dataclasses

import jax
import jax.numpy as jnp
from jax import lax
from jax.experimental import pallas as pl
from jax.experimental.pallas import tpu as pltpu
from jax.experimental.pallas import tpu_sc as plsc

VOCAB = 1000000
EMBED = 64
B = 4096
L = 50

NC = 2
NS = 16
NW = NC * NS
BW = B // NW
RW = BW * L
CB = 8
CHUNK = CB * L
NCHUNK = BW // CB
LPAD = 64
NG = L // 16


def _sc_kernel(center_hbm, ctx_hbm, ev_hbm, eu_hbm, out_hbm,
               cidx_v, v_rows, uidx_v, u_rows, s_tile, o_all, sem):
    wid = lax.axis_index("s") * NC + lax.axis_index("c")
    iota = lax.iota(jnp.int32, 16)

    pltpu.sync_copy(center_hbm.at[pl.ds(wid * BW, BW)], cidx_v)
    pltpu.sync_copy(ctx_hbm.at[pl.ds(wid * RW, RW)], uidx_v)

    pltpu.async_copy(ev_hbm.at[cidx_v], v_rows, sem).wait()

    def dot_row(r, v0, v1, v2, v3):
        acc = u_rows[r, pl.ds(0, 16)] * v0
        acc = acc + u_rows[r, pl.ds(16, 16)] * v1
        acc = acc + u_rows[r, pl.ds(32, 16)] * v2
        acc = acc + u_rows[r, pl.ds(48, 16)] * v3
        return acc

    def reduce_tile():
        out16 = s_tile[0, pl.ds(0, 16)]
        for k in range(1, 16):
            out16 = out16 + s_tile[k, pl.ds(0, 16)]
        return out16

    @pl.loop(0, NCHUNK)
    def _(c):
        pltpu.async_copy(
            eu_hbm.at[uidx_v.at[pl.ds(c * CHUNK, CHUNK)]], u_rows, sem
        ).wait()

        for b in range(CB):
            bb = c * CB + b
            v0 = v_rows[bb, pl.ds(0, 16)]
            v1 = v_rows[bb, pl.ds(16, 16)]
            v2 = v_rows[bb, pl.ds(32, 16)]
            v3 = v_rows[bb, pl.ds(48, 16)]
            for g in range(NG):
                for j in range(16):
                    acc = dot_row(b * L + 16 * g + j, v0, v1, v2, v3)
                    plsc.store_scatter(s_tile, [iota, iota * 0 + j], acc)
                o16 = reduce_tile()
                o_all[pl.ds(bb * LPAD + 16 * g, 16)] = o16

        for j in range(16):
            b = j // 2
            if j % 2 == 0:
                tv0 = v_rows[c * CB + b, pl.ds(0, 16)]
                tv1 = v_rows[c * CB + b, pl.ds(16, 16)]
                tv2 = v_rows[c * CB + b, pl.ds(32, 16)]
                tv3 = v_rows[c * CB + b, pl.ds(48, 16)]
            acc = dot_row(b * L + 48 + (j % 2), tv0, tv1, tv2, tv3)
            plsc.store_scatter(s_tile, [iota, iota * 0 + j], acc)
        o16 = reduce_tile()
        dest = (c * CB + (iota // 2)) * LPAD + 48 + (iota % 2)
        plsc.store_scatter(o_all, [dest], o16)

    pltpu.sync_copy(o_all, out_hbm.at[pl.ds(wid * BW * LPAD, BW * LPAD)])


def kernel(center, context_negative, embed_v, embed_u):
    mesh = plsc.VectorSubcoreMesh(core_axis_name="c", subcore_axis_name="s")
    cp = pltpu.CompilerParams()
    if "needs_layout_passes" in pltpu.CompilerParams.__dataclass_fields__:
        cp = dataclasses.replace(cp, needs_layout_passes=False)
    if "use_tc_tiling_on_sc" in pltpu.CompilerParams.__dataclass_fields__:
        cp = dataclasses.replace(cp, use_tc_tiling_on_sc=False)
    k = pl.kernel(
        _sc_kernel,
        compiler_params=cp,
        out_type=jax.ShapeDtypeStruct((B * LPAD,), jnp.float32),
        mesh=mesh,
        scratch_types=[
            pltpu.VMEM((BW,), jnp.int32),
            pltpu.VMEM((BW, EMBED), jnp.float32),
            pltpu.VMEM((RW,), jnp.int32),
            pltpu.VMEM((CHUNK, EMBED), jnp.float32),
            pltpu.VMEM((16, 16), jnp.float32),
            pltpu.VMEM((BW * LPAD,), jnp.float32),
            pltpu.SemaphoreType.DMA,
        ],
    )
    out = k(center.reshape(B), context_negative.reshape(B * L), embed_v, embed_u)
    return out.reshape(B, LPAD)[:, :L].reshape(B, 1, L)
